# baseline (speedup 1.0000x reference)
import jax
import jax.numpy as jnp
from jax import lax
from jax.experimental import pallas as pl
from jax.experimental.pallas import tpu as pltpu

N_ROWS = 1024
N_COLS = 100000

_BC = 2048
_NBLK = (N_COLS + _BC - 1) // _BC
_LANES = 128


def _count_body(x_ref, v_ref, t_ref, out_ref, acc_ref):
    i = pl.program_id(0)

    @pl.when(i == 0)
    def _():
        acc_ref[...] = jnp.zeros_like(acc_ref)

    x = x_ref[...]
    v = v_ref[...]
    t_loc = t_ref[...] - i * _BC
    n_loc = N_COLS - i * _BC
    lane = lax.broadcasted_iota(jnp.int32, (N_ROWS, _BC), 1)
    contrib = ((x > v) & (lane < n_loc)) | ((x == v) & (lane < t_loc))
    c = contrib.astype(jnp.float32)
    partial = c[:, 0:_LANES]
    for s in range(1, _BC // _LANES):
        partial = partial + c[:, s * _LANES:(s + 1) * _LANES]
    acc_ref[...] += partial

    @pl.when(i == _NBLK - 1)
    def _():
        rank = jnp.sum(acc_ref[...], axis=1, keepdims=True)
        top1 = jnp.sum((rank < 0.5).astype(jnp.float32))
        top5 = jnp.sum((rank < 4.5).astype(jnp.float32))
        out_ref[...] = jnp.concatenate(
            [top1.reshape(1, 1), top5.reshape(1, 1)], axis=1
        ) * (100.0 / N_ROWS)


@jax.jit
def kernel(pred, target):
    t2 = target.astype(jnp.int32).reshape(N_ROWS, 1)
    v2 = jnp.zeros((N_ROWS, 1), jnp.float32)
    out = pl.pallas_call(
        _count_body,
        grid=(_NBLK,),
        in_specs=[
            pl.BlockSpec((N_ROWS, _BC), lambda i: (0, i)),
            pl.BlockSpec((N_ROWS, 1), lambda i: (0, 0)),
            pl.BlockSpec((N_ROWS, 1), lambda i: (0, 0)),
        ],
        out_specs=pl.BlockSpec((1, 2), lambda i: (0, 0)),
        out_shape=jax.ShapeDtypeStruct((1, 2), jnp.float32),
        scratch_shapes=[pltpu.VMEM((N_ROWS, _LANES), jnp.float32)],
    )(pred, v2, t2)
    return out.reshape(2)


# speedup vs baseline: 1.0049x; 1.0049x over previous
import jax
import jax.numpy as jnp
from jax import lax
from jax.experimental import pallas as pl
from jax.experimental.pallas import tpu as pltpu

N_ROWS = 1024
N_COLS = 100000

_BR = 32
_NBLK = N_ROWS // _BR


def _body(x_ref, t_ref, out_ref):
    i = pl.program_id(0)
    x = x_ref[...]
    t = t_ref[...]
    col = lax.broadcasted_iota(jnp.int32, (_BR, N_COLS), 1)
    at_t = col == t
    v = jnp.max(jnp.where(at_t, x, -jnp.inf), axis=1, keepdims=True)
    contrib = (x > v) | ((x == v) & (col < t))
    rank = jnp.sum(contrib.astype(jnp.float32), axis=1, keepdims=True)
    top1 = jnp.sum((rank < 0.5).astype(jnp.float32))
    top5 = jnp.sum((rank < 4.5).astype(jnp.float32))
    part = jnp.concatenate(
        [top1.reshape(1, 1), top5.reshape(1, 1)], axis=1
    ) * (100.0 / N_ROWS)

    @pl.when(i == 0)
    def _():
        out_ref[...] = part

    @pl.when(i > 0)
    def _():
        out_ref[...] += part


@jax.jit
def kernel(pred, target):
    t2 = target.astype(jnp.int32).reshape(N_ROWS, 1)
    out = pl.pallas_call(
        _body,
        grid=(_NBLK,),
        in_specs=[
            pl.BlockSpec((_BR, N_COLS), lambda i: (i, 0)),
            pl.BlockSpec((_BR, 1), lambda i: (i, 0)),
        ],
        out_specs=pl.BlockSpec((1, 2), lambda i: (0, 0)),
        out_shape=jax.ShapeDtypeStruct((1, 2), jnp.float32),
    )(pred, t2)
    return out.reshape(2)
